# baseline (speedup 1.0000x reference)
import jax
import jax.numpy as jnp
from jax.experimental import pallas as pl

_B, _G, _H = 8, 64, 256
_P = _B * _G


def _dot(a, b, precision=jax.lax.Precision.DEFAULT):
    return jax.lax.dot_general(
        a, b, (((1,), (0,)), ((), ())),
        precision=precision,
        preferred_element_type=jnp.float32)


def _relu(x):
    return jnp.maximum(x, 0.0)


def _fused_kernel(pc_ref, fts_ref, grid_ref,
                  pe_W1, pe_b1, pe_W2, pe_b2, pe_W3, pe_b3,
                  fe_W1, fe_b1, fe_W2, fe_b2,
                  rn_W1, rn_b1, rn_W2, rn_b2,
                  cl_W1, cl_b1, cl_W2, cl_b2,
                  po_W1, po_b1, po_W2, po_b2,
                  sz_W1, sz_b1, sz_W2, sz_b2,
                  probs_ref, pose_ref, size_ref, proc_ref, gf_ref):
    pts = pc_ref[...].reshape(_P, 3)
    fts = fts_ref[...].reshape(_P, 64)
    pf = _relu(_dot(pts, pe_W1[...]) + pe_b1[...])
    pf = _relu(_dot(pf, pe_W2[...]) + pe_b2[...])
    pf = _dot(pf, pe_W3[...]) + pe_b3[...]
    fe = _relu(_dot(fts, fe_W1[...]) + fe_b1[...])
    fe = _dot(fe, fe_W2[...]) + fe_b2[...]
    combined = jnp.concatenate([pf, fe], axis=1)

    ptsT = pts.T
    gx = grid_ref[:, 0:1]
    gy = grid_ref[:, 1:2]
    gz = grid_ref[:, 2:3]
    dx = gx - ptsT[0:1, :]
    dy = gy - ptsT[1:2, :]
    dz = gz - ptsT[2:3, :]
    d = jnp.sqrt(dx * dx + dy * dy + dz * dz)
    dmin = jnp.min(d, axis=0, keepdims=True)
    j_iota = jax.lax.broadcasted_iota(jnp.int32, (_G, _P), 0)
    idx = jnp.min(jnp.where(d == dmin, j_iota, _G), axis=0, keepdims=True)

    q2 = jax.lax.broadcasted_iota(jnp.int32, (_P, _P), 0)
    p2 = jax.lax.broadcasted_iota(jnp.int32, (_P, _P), 1)
    cond = ((q2 >> 6) == (p2 >> 6)) & (idx == (q2 & (_G - 1)))
    win = jnp.max(jnp.where(cond, p2, -1), axis=1, keepdims=True)
    onehot = (p2 == win).astype(jnp.float32)
    gff = _dot(onehot, combined, jax.lax.Precision.HIGHEST)

    h = _relu(_dot(gff, rn_W1[...]) + rn_b1[...])
    procf = _dot(h, rn_W2[...]) + rn_b2[...]

    for b in range(_B):
        rows = slice(b * _G, (b + 1) * _G)
        gf_ref[b] = gff[rows, :].T
        proc_ref[b] = procf[rows, :].T

    bq = jax.lax.broadcasted_iota(jnp.int32, (_B, _P), 0)
    bp = jax.lax.broadcasted_iota(jnp.int32, (_B, _P), 1) >> 6
    avg = jnp.where(bq == bp, 1.0 / _G, 0.0).astype(jnp.float32)
    agg = _dot(avg, procf, jax.lax.Precision.HIGHEST)

    c1 = _relu(_dot(agg, cl_W1[...]) + cl_b1[...])
    logits = _dot(c1, cl_W2[...]) + cl_b2[...]
    m = jnp.max(logits, axis=1, keepdims=True)
    e = jnp.exp(logits - m)
    probs_ref[...] = e / jnp.sum(e, axis=1, keepdims=True)
    p1 = _relu(_dot(agg, po_W1[...]) + po_b1[...])
    pose_ref[...] = _dot(p1, po_W2[...]) + po_b2[...]
    s1 = _relu(_dot(agg, sz_W1[...]) + sz_b1[...])
    size_ref[...] = jax.nn.sigmoid(_dot(s1, sz_W2[...]) + sz_b2[...])


def kernel(point_cloud, features, grid_points,
           pe_W1, pe_b1, pe_W2, pe_b2, pe_W3, pe_b3,
           fe_W1, fe_b1, fe_W2, fe_b2,
           rn_W1, rn_b1, rn_W2, rn_b2,
           cl_W1, cl_b1, cl_W2, cl_b2,
           po_W1, po_b1, po_W2, po_b2,
           sz_W1, sz_b1, sz_W2, sz_b2):
    C = cl_W2.shape[1]
    biases = [pe_b1, pe_b2, pe_b3, fe_b1, fe_b2, rn_b1, rn_b2,
              cl_b1, cl_b2, po_b1, po_b2, sz_b1, sz_b2]
    (pe_b1, pe_b2, pe_b3, fe_b1, fe_b2, rn_b1, rn_b2,
     cl_b1, cl_b2, po_b1, po_b2, sz_b1, sz_b2) = [
        b.reshape(1, -1) for b in biases]

    full = lambda a: pl.BlockSpec(a.shape, lambda i: (0,) * a.ndim)
    in_specs = [
        pl.BlockSpec((_B, _G, 3), lambda i: (0, 0, 0)),
        pl.BlockSpec((_B, _G, 64), lambda i: (0, 0, 0)),
    ] + [full(a) for a in (
        grid_points,
        pe_W1, pe_b1, pe_W2, pe_b2, pe_W3, pe_b3,
        fe_W1, fe_b1, fe_W2, fe_b2,
        rn_W1, rn_b1, rn_W2, rn_b2,
        cl_W1, cl_b1, cl_W2, cl_b2,
        po_W1, po_b1, po_W2, po_b2,
        sz_W1, sz_b1, sz_W2, sz_b2)]

    out_shape = (
        jax.ShapeDtypeStruct((_B, C), jnp.float32),
        jax.ShapeDtypeStruct((_B, 7), jnp.float32),
        jax.ShapeDtypeStruct((_B, 3), jnp.float32),
        jax.ShapeDtypeStruct((_B, _H, _G), jnp.float32),
        jax.ShapeDtypeStruct((_B, 2 * _H, _G), jnp.float32),
    )
    out_specs = (
        pl.BlockSpec((_B, C), lambda i: (0, 0)),
        pl.BlockSpec((_B, 7), lambda i: (0, 0)),
        pl.BlockSpec((_B, 3), lambda i: (0, 0)),
        pl.BlockSpec((_B, _H, _G), lambda i: (0, 0, 0)),
        pl.BlockSpec((_B, 2 * _H, _G), lambda i: (0, 0, 0)),
    )
    return pl.pallas_call(
        _fused_kernel, out_shape=out_shape, grid=(1,),
        in_specs=in_specs, out_specs=out_specs)(
            point_cloud, features, grid_points,
            pe_W1, pe_b1, pe_W2, pe_b2, pe_W3, pe_b3,
            fe_W1, fe_b1, fe_W2, fe_b2,
            rn_W1, rn_b1, rn_W2, rn_b2,
            cl_W1, cl_b1, cl_W2, cl_b2,
            po_W1, po_b1, po_W2, po_b2,
            sz_W1, sz_b1, sz_W2, sz_b2)


# speedup vs baseline: 1.7816x; 1.7816x over previous
import jax
import jax.numpy as jnp
from jax.experimental import pallas as pl

_B, _G, _H = 8, 64, 256
_P = _B * _G


def _dot(a, b, precision=jax.lax.Precision.DEFAULT):
    return jax.lax.dot_general(
        a, b, (((1,), (0,)), ((), ())),
        precision=precision,
        preferred_element_type=jnp.float32)


def _relu(x):
    return jnp.maximum(x, 0.0)


def _fused_kernel(pc_ref, fts_ref, grid_ref,
                  pe_W1, pe_b1, pe_W2, pe_b2, pe_W3, pe_b3,
                  fe_W1, fe_b1, fe_W2, fe_b2,
                  rn_W1, rn_b1, rn_W2, rn_b2,
                  cl_W1, cl_b1, cl_W2, cl_b2,
                  po_W1, po_b1, po_W2, po_b2,
                  sz_W1, sz_b1, sz_W2, sz_b2,
                  probs_ref, pose_ref, size_ref, proc_ref, gf_ref):
    pts = pc_ref[...]
    fts = fts_ref[...]
    pf = _relu(_dot(pts, pe_W1[...]) + pe_b1[...])
    pf = _relu(_dot(pf, pe_W2[...]) + pe_b2[...])
    pf = _dot(pf, pe_W3[...]) + pe_b3[...]
    fe = _relu(_dot(fts, fe_W1[...]) + fe_b1[...])
    fe = _dot(fe, fe_W2[...]) + fe_b2[...]
    combined = jnp.concatenate([pf, fe], axis=1)

    ptsT = pts.T
    gx = grid_ref[:, 0:1]
    gy = grid_ref[:, 1:2]
    gz = grid_ref[:, 2:3]
    dx = gx - ptsT[0:1, :]
    dy = gy - ptsT[1:2, :]
    dz = gz - ptsT[2:3, :]
    d = jnp.sqrt(dx * dx + dy * dy + dz * dz)
    dmin = jnp.min(d, axis=0, keepdims=True)
    j_iota = jax.lax.broadcasted_iota(jnp.int32, (_G, _P), 0)
    idx = jnp.min(jnp.where(d == dmin, j_iota, _G), axis=0, keepdims=True)

    q2 = jax.lax.broadcasted_iota(jnp.int32, (_P, _P), 0)
    p2 = jax.lax.broadcasted_iota(jnp.int32, (_P, _P), 1)
    cond = ((q2 >> 6) == (p2 >> 6)) & (idx == (q2 & (_G - 1)))
    win = jnp.max(jnp.where(cond, p2, -1), axis=1, keepdims=True)
    onehot = (p2 == win).astype(jnp.float32)
    gff = _dot(onehot, combined, jax.lax.Precision.HIGHEST)

    h = _relu(_dot(gff, rn_W1[...]) + rn_b1[...])
    procf = _dot(h, rn_W2[...]) + rn_b2[...]

    for b in range(_B):
        rows = slice(b * _G, (b + 1) * _G)
        gf_ref[b] = gff[rows, :].T
        proc_ref[b] = procf[rows, :].T

    bq = jax.lax.broadcasted_iota(jnp.int32, (_B, _P), 0)
    bp = jax.lax.broadcasted_iota(jnp.int32, (_B, _P), 1) >> 6
    avg = jnp.where(bq == bp, 1.0 / _G, 0.0).astype(jnp.float32)
    agg = _dot(avg, procf, jax.lax.Precision.HIGHEST)

    c1 = _relu(_dot(agg, cl_W1[...]) + cl_b1[...])
    logits = _dot(c1, cl_W2[...]) + cl_b2[...]
    m = jnp.max(logits, axis=1, keepdims=True)
    e = jnp.exp(logits - m)
    probs_ref[...] = e / jnp.sum(e, axis=1, keepdims=True)
    p1 = _relu(_dot(agg, po_W1[...]) + po_b1[...])
    pose_ref[...] = _dot(p1, po_W2[...]) + po_b2[...]
    s1 = _relu(_dot(agg, sz_W1[...]) + sz_b1[...])
    size_ref[...] = jax.nn.sigmoid(_dot(s1, sz_W2[...]) + sz_b2[...])


def kernel(point_cloud, features, grid_points,
           pe_W1, pe_b1, pe_W2, pe_b2, pe_W3, pe_b3,
           fe_W1, fe_b1, fe_W2, fe_b2,
           rn_W1, rn_b1, rn_W2, rn_b2,
           cl_W1, cl_b1, cl_W2, cl_b2,
           po_W1, po_b1, po_W2, po_b2,
           sz_W1, sz_b1, sz_W2, sz_b2):
    C = cl_W2.shape[1]
    pts = point_cloud[:, :_G, :].reshape(_P, 3)
    fts = features[:, :_G, :].reshape(_P, 64)
    biases = [pe_b1, pe_b2, pe_b3, fe_b1, fe_b2, rn_b1, rn_b2,
              cl_b1, cl_b2, po_b1, po_b2, sz_b1, sz_b2]
    (pe_b1, pe_b2, pe_b3, fe_b1, fe_b2, rn_b1, rn_b2,
     cl_b1, cl_b2, po_b1, po_b2, sz_b1, sz_b2) = [
        b.reshape(1, -1) for b in biases]

    out_shape = (
        jax.ShapeDtypeStruct((_B, C), jnp.float32),
        jax.ShapeDtypeStruct((_B, 7), jnp.float32),
        jax.ShapeDtypeStruct((_B, 3), jnp.float32),
        jax.ShapeDtypeStruct((_B, _H, _G), jnp.float32),
        jax.ShapeDtypeStruct((_B, 2 * _H, _G), jnp.float32),
    )
    return pl.pallas_call(
        _fused_kernel, out_shape=out_shape)(
            pts, fts, grid_points,
            pe_W1, pe_b1, pe_W2, pe_b2, pe_W3, pe_b3,
            fe_W1, fe_b1, fe_W2, fe_b2,
            rn_W1, rn_b1, rn_W2, rn_b2,
            cl_W1, cl_b1, cl_W2, cl_b2,
            po_W1, po_b1, po_W2, po_b2,
            sz_W1, sz_b1, sz_W2, sz_b2)
